# baseline (speedup 1.0000x reference)
import functools

import jax
import jax.numpy as jnp
from jax import lax
from jax.experimental import pallas as pl
from jax.experimental.pallas import tpu as pltpu
from jax.experimental.pallas import tpu_sc as plsc

NUM_WORKERS = 32
NUM_CHUNKS = 4
CHUNK = 128
LANES = 16
EMBED = 128


def _body(uids_hbm, iids_hbm, utab_hbm, itab_hbm, ubias_hbm, ibias_hbm,
          out_hbm, uid_v, iid_v, urows, irows, ubv, ibv, sums,
          sem_u, sem_i, sem_ub, sem_ib):
    wid = lax.axis_index("s") * 2 + lax.axis_index("c")
    pltpu.sync_copy(uids_hbm.at[wid], uid_v)
    pltpu.sync_copy(iids_hbm.at[wid], iid_v)
    riota = lax.iota(jnp.int32, LANES)

    for c in range(NUM_CHUNKS):
        cu = pltpu.async_copy(utab_hbm.at[uid_v.at[c]], urows, sem_u)
        ci = pltpu.async_copy(itab_hbm.at[iid_v.at[c]], irows, sem_i)
        cub = pltpu.async_copy(ubias_hbm.at[uid_v.at[c]], ubv, sem_ub)
        cib = pltpu.async_copy(ibias_hbm.at[iid_v.at[c]], ibv, sem_ib)
        cu.wait()
        ci.wait()
        cub.wait()
        cib.wait()

        for g in range(CHUNK // LANES):
            rid = riota + (g * LANES)

            def dbody(d, acc, rid=rid):
                cid = jnp.full((LANES,), 0, jnp.int32) + d
                u = plsc.load_gather(urows, [rid, cid])
                i = plsc.load_gather(irows, [rid, cid])
                return acc + u * i

            acc = lax.fori_loop(0, EMBED, dbody, jnp.zeros((LANES,), jnp.float32))
            acc = acc + ubv[pl.ds(g * LANES, LANES)] + ibv[pl.ds(g * LANES, LANES)]
            sums[pl.ds(g * LANES, LANES)] = acc

        pltpu.sync_copy(sums, out_hbm.at[wid, c])


@jax.jit
def _sc_call(uids, iids, utab, itab, ubias, ibias):
    mesh = plsc.VectorSubcoreMesh(core_axis_name="c", subcore_axis_name="s")
    return pl.kernel(
        _body,
        out_type=jax.ShapeDtypeStruct((NUM_WORKERS, NUM_CHUNKS, CHUNK), jnp.float32),
        mesh=mesh,
        compiler_params=pltpu.CompilerParams(needs_layout_passes=False),
        scratch_types=[
            pltpu.VMEM((NUM_CHUNKS, CHUNK), jnp.int32),
            pltpu.VMEM((NUM_CHUNKS, CHUNK), jnp.int32),
            pltpu.VMEM((CHUNK, EMBED), jnp.float32),
            pltpu.VMEM((CHUNK, EMBED), jnp.float32),
            pltpu.VMEM((CHUNK,), jnp.float32),
            pltpu.VMEM((CHUNK,), jnp.float32),
            pltpu.VMEM((CHUNK,), jnp.float32),
            pltpu.SemaphoreType.DMA,
            pltpu.SemaphoreType.DMA,
            pltpu.SemaphoreType.DMA,
            pltpu.SemaphoreType.DMA,
        ],
    )(uids, iids, utab, itab, ubias, ibias)


def kernel(user_ids, item_ids, user_table, item_table, user_bias, item_bias):
    batch = user_ids.shape[0]
    uids = user_ids.astype(jnp.int32).reshape(NUM_WORKERS, NUM_CHUNKS, CHUNK)
    iids = item_ids.astype(jnp.int32).reshape(NUM_WORKERS, NUM_CHUNKS, CHUNK)
    out = _sc_call(uids, iids, user_table, item_table,
                   user_bias.reshape(-1), item_bias.reshape(-1))
    return out.reshape(batch, 1)


# speedup vs baseline: 1.0319x; 1.0319x over previous
import jax
import jax.numpy as jnp
from jax import lax
from jax.experimental import pallas as pl
from jax.experimental.pallas import tpu as pltpu
from jax.experimental.pallas import tpu_sc as plsc

NUM_WORKERS = 32
NUM_CHUNKS = 4
CHUNK = 128
LANES = 16
EMBED = 128
UNROLL = 8


def _body(uids_hbm, iids_hbm, utab_hbm, itab_hbm, ubias_hbm, ibias_hbm,
          out_hbm, uid_v, iid_v, urows0, urows1, irows0, irows1,
          ubv0, ubv1, ibv0, ibv1, sums,
          sem_u0, sem_u1, sem_i0, sem_i1, sem_b0, sem_b1):
    wid = lax.axis_index("s") * 2 + lax.axis_index("c")
    pltpu.sync_copy(uids_hbm.at[wid], uid_v)
    pltpu.sync_copy(iids_hbm.at[wid], iid_v)
    riota = lax.iota(jnp.int32, LANES)

    urows = (urows0, urows1)
    irows = (irows0, irows1)
    ubv = (ubv0, ubv1)
    ibv = (ibv0, ibv1)
    sem_u = (sem_u0, sem_u1)
    sem_i = (sem_i0, sem_i1)
    sem_b = (sem_b0, sem_b1)

    def start(c, b):
        return (
            pltpu.async_copy(utab_hbm.at[uid_v.at[c]], urows[b], sem_u[b]),
            pltpu.async_copy(itab_hbm.at[iid_v.at[c]], irows[b], sem_i[b]),
            pltpu.async_copy(ubias_hbm.at[uid_v.at[c]], ubv[b], sem_b[b]),
            pltpu.async_copy(ibias_hbm.at[iid_v.at[c]], ibv[b], sem_b[b]),
        )

    cps = start(0, 0)
    for c in range(NUM_CHUNKS):
        b = c % 2
        nxt = start(c + 1, 1 - b) if c + 1 < NUM_CHUNKS else None
        for cp in cps:
            cp.wait()

        for g in range(CHUNK // LANES):
            rid = riota + (g * LANES)

            def dbody(j, acc, rid=rid, b=b):
                base = j * UNROLL
                for k in range(UNROLL):
                    cid = jnp.full((LANES,), 0, jnp.int32) + (base + k)
                    u = plsc.load_gather(urows[b], [rid, cid])
                    i = plsc.load_gather(irows[b], [rid, cid])
                    acc = acc + u * i
                return acc

            acc = lax.fori_loop(0, EMBED // UNROLL, dbody,
                                jnp.zeros((LANES,), jnp.float32))
            acc = acc + ubv[b][pl.ds(g * LANES, LANES)] + ibv[b][pl.ds(g * LANES, LANES)]
            sums[pl.ds(g * LANES, LANES)] = acc

        pltpu.sync_copy(sums, out_hbm.at[wid, c])
        cps = nxt


@jax.jit
def _sc_call(uids, iids, utab, itab, ubias, ibias):
    mesh = plsc.VectorSubcoreMesh(core_axis_name="c", subcore_axis_name="s")
    return pl.kernel(
        _body,
        out_type=jax.ShapeDtypeStruct((NUM_WORKERS, NUM_CHUNKS, CHUNK), jnp.float32),
        mesh=mesh,
        compiler_params=pltpu.CompilerParams(needs_layout_passes=False),
        scratch_types=[
            pltpu.VMEM((NUM_CHUNKS, CHUNK), jnp.int32),
            pltpu.VMEM((NUM_CHUNKS, CHUNK), jnp.int32),
            pltpu.VMEM((CHUNK, EMBED), jnp.float32),
            pltpu.VMEM((CHUNK, EMBED), jnp.float32),
            pltpu.VMEM((CHUNK, EMBED), jnp.float32),
            pltpu.VMEM((CHUNK, EMBED), jnp.float32),
            pltpu.VMEM((CHUNK,), jnp.float32),
            pltpu.VMEM((CHUNK,), jnp.float32),
            pltpu.VMEM((CHUNK,), jnp.float32),
            pltpu.VMEM((CHUNK,), jnp.float32),
            pltpu.VMEM((CHUNK,), jnp.float32),
            pltpu.SemaphoreType.DMA,
            pltpu.SemaphoreType.DMA,
            pltpu.SemaphoreType.DMA,
            pltpu.SemaphoreType.DMA,
            pltpu.SemaphoreType.DMA,
            pltpu.SemaphoreType.DMA,
        ],
    )(uids, iids, utab, itab, ubias, ibias)


def kernel(user_ids, item_ids, user_table, item_table, user_bias, item_bias):
    batch = user_ids.shape[0]
    uids = user_ids.astype(jnp.int32).reshape(NUM_WORKERS, NUM_CHUNKS, CHUNK)
    iids = item_ids.astype(jnp.int32).reshape(NUM_WORKERS, NUM_CHUNKS, CHUNK)
    out = _sc_call(uids, iids, user_table, item_table,
                   user_bias.reshape(-1), item_bias.reshape(-1))
    return out.reshape(batch, 1)


# speedup vs baseline: 1.4927x; 1.4466x over previous
import jax
import jax.numpy as jnp
from jax import lax
from jax.experimental import pallas as pl
from jax.experimental.pallas import tpu as pltpu
from jax.experimental.pallas import tpu_sc as plsc

NUM_WORKERS = 32
NUM_SUBCORES = 16
NUM_CHUNKS = 4
CHUNK = 128
LANES = 16
EMBED = 128
UNROLL = 8
NUM_USERS_ = 100000
NUM_ITEMS_ = 1000000


def _body(uids_hbm, iids_hbm, utab_hbm, itab_hbm, ubias_hbm, ibias_hbm,
          out_hbm, uid_v, iid_v, urows0, urows1, irows0, irows1,
          ubv, ibv, sums,
          sem_u0, sem_u1, sem_i0, sem_i1, sem_b):
    cid_ax = lax.axis_index("c")
    sid = lax.axis_index("s")
    wid = sid * 2 + cid_ax
    pltpu.sync_copy(uids_hbm.at[wid], uid_v)
    pltpu.sync_copy(iids_hbm.at[wid], iid_v)

    riota = lax.iota(jnp.int32, LANES)
    urows = (urows0, urows1)
    irows = (irows0, irows1)
    sem_u = (sem_u0, sem_u1)
    sem_i = (sem_i0, sem_i1)

    def start(c, b):
        return (
            pltpu.async_copy(utab_hbm.at[uid_v.at[c]], urows[b], sem_u[b]),
            pltpu.async_copy(itab_hbm.at[iid_v.at[c]], irows[b], sem_i[b]),
        )

    cps = start(0, 0)
    for c in range(NUM_CHUNKS):
        b = c % 2
        nxt = start(c + 1, 1 - b) if c + 1 < NUM_CHUNKS else None

        cb_u = pltpu.async_copy(ubias_hbm.at[0].at[uid_v.at[c]], ubv, sem_b)
        cb_i = pltpu.async_copy(ibias_hbm.at[0].at[iid_v.at[c]], ibv, sem_b)
        for cp in cps:
            cp.wait()

        for g in range(CHUNK // LANES):
            rid = riota + (g * LANES)

            def dbody(j, acc, rid=rid, b=b):
                base = j * UNROLL
                for k in range(UNROLL):
                    cid = jnp.full((LANES,), 0, jnp.int32) + (base + k)
                    u = plsc.load_gather(urows[b], [rid, cid])
                    i = plsc.load_gather(irows[b], [rid, cid])
                    acc = acc + u * i
                return acc

            acc = lax.fori_loop(0, EMBED // UNROLL, dbody,
                                jnp.zeros((LANES,), jnp.float32))
            if g == 0:
                cb_u.wait()
                cb_i.wait()
            acc = acc + ubv[pl.ds(g * LANES, LANES)] + ibv[pl.ds(g * LANES, LANES)]
            sums[pl.ds(g * LANES, LANES)] = acc

        pltpu.sync_copy(sums, out_hbm.at[wid, c])
        cps = nxt


@jax.jit
def _sc_call(uids, iids, utab, itab, ubias, ibias):
    mesh = plsc.VectorSubcoreMesh(core_axis_name="c", subcore_axis_name="s")
    return pl.kernel(
        _body,
        out_type=jax.ShapeDtypeStruct((NUM_WORKERS, NUM_CHUNKS, CHUNK), jnp.float32),
        mesh=mesh,
        compiler_params=pltpu.CompilerParams(needs_layout_passes=False),
        scratch_types=[
            pltpu.VMEM((NUM_CHUNKS, CHUNK), jnp.int32),
            pltpu.VMEM((NUM_CHUNKS, CHUNK), jnp.int32),
            pltpu.VMEM((CHUNK, EMBED), jnp.float32),
            pltpu.VMEM((CHUNK, EMBED), jnp.float32),
            pltpu.VMEM((CHUNK, EMBED), jnp.float32),
            pltpu.VMEM((CHUNK, EMBED), jnp.float32),
            pltpu.VMEM((CHUNK,), jnp.float32),
            pltpu.VMEM((CHUNK,), jnp.float32),
            pltpu.VMEM((CHUNK,), jnp.float32),
            pltpu.SemaphoreType.DMA,
            pltpu.SemaphoreType.DMA,
            pltpu.SemaphoreType.DMA,
            pltpu.SemaphoreType.DMA,
            pltpu.SemaphoreType.DMA,
        ],
    )(uids, iids, utab, itab, ubias, ibias)


def kernel(user_ids, item_ids, user_table, item_table, user_bias, item_bias):
    batch = user_ids.shape[0]
    uids = user_ids.astype(jnp.int32).reshape(NUM_WORKERS, NUM_CHUNKS, CHUNK)
    iids = item_ids.astype(jnp.int32).reshape(NUM_WORKERS, NUM_CHUNKS, CHUNK)
    out = _sc_call(uids, iids, user_table, item_table,
                   jnp.transpose(user_bias, (1, 0)),
                   jnp.transpose(item_bias, (1, 0)))
    return out.reshape(batch, 1)


# speedup vs baseline: 2.3848x; 1.5976x over previous
import jax
import jax.numpy as jnp
from jax import lax
from jax.experimental import pallas as pl
from jax.experimental.pallas import tpu as pltpu
from jax.experimental.pallas import tpu_sc as plsc

NUM_WORKERS = 32
NUM_SUBCORES = 16
NUM_CHUNKS = 4
CHUNK = 128
LANES = 16
EMBED = 128
UNROLL = 8
NUM_USERS_ = 100000
NUM_ITEMS_ = 1000000


def _body(uids_hbm, iids_hbm, utab_hbm, itab_hbm, ubias_hbm, ibias_hbm,
          out_hbm, uid_v, iid_v, urows0, urows1, irows0, irows1,
          ubv, ibv, sums,
          sem_u0, sem_u1, sem_i0, sem_i1, sem_b):
    cid_ax = lax.axis_index("c")
    sid = lax.axis_index("s")
    wid = sid * 2 + cid_ax
    pltpu.sync_copy(uids_hbm.at[wid], uid_v)
    pltpu.sync_copy(iids_hbm.at[wid], iid_v)

    riota = lax.iota(jnp.int32, LANES)
    urows = (urows0, urows1)
    irows = (irows0, irows1)
    sem_u = (sem_u0, sem_u1)
    sem_i = (sem_i0, sem_i1)

    def start(c, b):
        return (
            pltpu.async_copy(utab_hbm.at[uid_v.at[c]], urows[b], sem_u[b]),
            pltpu.async_copy(itab_hbm.at[iid_v.at[c]], irows[b], sem_i[b]),
        )

    cps = start(0, 0)
    for c in range(NUM_CHUNKS):
        b = c % 2
        nxt = start(c + 1, 1 - b) if c + 1 < NUM_CHUNKS else None

        cb_u = pltpu.async_copy(ubias_hbm.at[0].at[uid_v.at[c]], ubv, sem_b)
        cb_i = pltpu.async_copy(ibias_hbm.at[0].at[iid_v.at[c]], ibv, sem_b)
        for cp in cps:
            cp.wait()

        def gbody(g, carry, b=b):
            accs = []
            for r in range(LANES):
                row = g * LANES + r
                prods = [urows[b][row, pl.ds(k * LANES, LANES)]
                         * irows[b][row, pl.ds(k * LANES, LANES)]
                         for k in range(EMBED // LANES)]
                while len(prods) > 1:
                    prods = [prods[i] + prods[i + 1]
                             for i in range(0, len(prods) - 1, 2)] + (
                                 [prods[-1]] if len(prods) % 2 else [])
                accs.append(prods[0])
            cur = accs
            for s in range(4):
                st = 1 << s
                mask = (riota & st) == 0
                perm = riota ^ st
                nxt = []
                for j in range(len(cur) // 2):
                    a, bb = cur[2 * j], cur[2 * j + 1]
                    a_sh = a.at[perm].get(mode="promise_in_bounds")
                    b_sh = bb.at[perm].get(mode="promise_in_bounds")
                    nxt.append(jnp.where(mask, a, b_sh) + jnp.where(mask, a_sh, bb))
                cur = nxt
            acc = (cur[0] + ubv[pl.ds(g * LANES, LANES)]
                   + ibv[pl.ds(g * LANES, LANES)])
            sums[pl.ds(g * LANES, LANES)] = acc
            return carry

        cb_u.wait()
        cb_i.wait()
        lax.fori_loop(0, CHUNK // LANES, gbody, 0)

        pltpu.sync_copy(sums, out_hbm.at[wid, c])
        cps = nxt


@jax.jit
def _sc_call(uids, iids, utab, itab, ubias, ibias):
    mesh = plsc.VectorSubcoreMesh(core_axis_name="c", subcore_axis_name="s")
    return pl.kernel(
        _body,
        out_type=jax.ShapeDtypeStruct((NUM_WORKERS, NUM_CHUNKS, CHUNK), jnp.float32),
        mesh=mesh,
        compiler_params=pltpu.CompilerParams(needs_layout_passes=False),
        scratch_types=[
            pltpu.VMEM((NUM_CHUNKS, CHUNK), jnp.int32),
            pltpu.VMEM((NUM_CHUNKS, CHUNK), jnp.int32),
            pltpu.VMEM((CHUNK, EMBED), jnp.float32),
            pltpu.VMEM((CHUNK, EMBED), jnp.float32),
            pltpu.VMEM((CHUNK, EMBED), jnp.float32),
            pltpu.VMEM((CHUNK, EMBED), jnp.float32),
            pltpu.VMEM((CHUNK,), jnp.float32),
            pltpu.VMEM((CHUNK,), jnp.float32),
            pltpu.VMEM((CHUNK,), jnp.float32),
            pltpu.SemaphoreType.DMA,
            pltpu.SemaphoreType.DMA,
            pltpu.SemaphoreType.DMA,
            pltpu.SemaphoreType.DMA,
            pltpu.SemaphoreType.DMA,
        ],
    )(uids, iids, utab, itab, ubias, ibias)


def kernel(user_ids, item_ids, user_table, item_table, user_bias, item_bias):
    batch = user_ids.shape[0]
    uids = user_ids.astype(jnp.int32).reshape(NUM_WORKERS, NUM_CHUNKS, CHUNK)
    iids = item_ids.astype(jnp.int32).reshape(NUM_WORKERS, NUM_CHUNKS, CHUNK)
    out = _sc_call(uids, iids, user_table, item_table,
                   jnp.transpose(user_bias, (1, 0)),
                   jnp.transpose(item_bias, (1, 0)))
    return out.reshape(batch, 1)
